# baseline (speedup 1.0000x reference)
import jax
import jax.numpy as jnp
from jax import lax
from jax.experimental import pallas as pl
from jax.experimental.pallas import tpu as pltpu
from jax.experimental.pallas import tpu_sc as plsc

VOCAB = 32000
HIDDEN = 4096
BATCH = 4
SEQ = 4096

NC = 2
NS = 16
NW = NC * NS

B = BATCH * SEQ
B_PER_W = B // NW
K = 4
NBUF = 4
RA = 2
NCHUNK = B_PER_W // K


def _gather_body(ids_hbm, table_hbm, out_hbm, idx_v, bufs, gsems, psems):
    wid = lax.axis_index("s") * NC + lax.axis_index("c")
    base = wid * B_PER_W

    pltpu.sync_copy(ids_hbm.at[pl.ds(wid * NCHUNK, NCHUNK)], idx_v)

    def gather_start(g, b):
        pltpu.async_copy(table_hbm.at[idx_v.at[g]], bufs[b], gsems[b])

    def gather_wait(b):
        pltpu.make_async_copy(table_hbm.at[idx_v.at[0]], bufs[b],
                              gsems[b]).wait()

    def put_start(g, b):
        pltpu.async_copy(bufs[b], out_hbm.at[pl.ds(base + g * K, K)], psems[b])

    def put_wait(b):
        pltpu.make_async_copy(bufs[b], out_hbm.at[pl.ds(base, K)],
                              psems[b]).wait()


    for i in range(RA):
        gather_start(i, i)
    for i in range(RA):
        gather_start(i + RA, (i + RA) % NBUF)
        gather_wait(i % NBUF)
        put_start(i, i % NBUF)

    @pl.loop(RA, NCHUNK - RA, step=NBUF)
    def _(i0):
        for j in range(NBUF):
            i = i0 + j
            b = (RA + j) % NBUF
            bnext = (RA + j + RA) % NBUF
            put_wait(bnext)
            gather_start(i + RA, bnext)
            gather_wait(b)
            put_start(i, b)

    for i in range(NCHUNK - RA, NCHUNK):
        b = i % NBUF
        put_wait((i + RA) % NBUF)
        gather_wait(b)
        put_start(i, b)
    for i in range(NCHUNK, NCHUNK + RA):
        put_wait((i + RA) % NBUF)


@jax.jit
def _embed_gather(ids, W):
    mesh = plsc.VectorSubcoreMesh(core_axis_name="c", subcore_axis_name="s")
    run = pl.kernel(
        _gather_body,
        out_type=jax.ShapeDtypeStruct((B, HIDDEN), jnp.float32),
        mesh=mesh,
        scratch_types=[
            pltpu.VMEM((NCHUNK, K), jnp.int32),
            [pltpu.VMEM((K, HIDDEN), jnp.float32) for _ in range(NBUF)],
            [pltpu.SemaphoreType.DMA for _ in range(NBUF)],
            [pltpu.SemaphoreType.DMA for _ in range(NBUF)],
        ],
    )
    return run(ids, W)


def kernel(inputs, W):
    ids = inputs[..., 0].reshape(B // K, K)
    attention_mask = inputs[..., 1]
    hidden = _embed_gather(ids, W)
    return hidden.reshape(BATCH, SEQ, HIDDEN), attention_mask


# speedup vs baseline: 1.0107x; 1.0107x over previous
import jax
import jax.numpy as jnp
from jax import lax
from jax.experimental import pallas as pl
from jax.experimental.pallas import tpu as pltpu
from jax.experimental.pallas import tpu_sc as plsc

VOCAB = 32000
HIDDEN = 4096
BATCH = 4
SEQ = 4096

NC = 2
NS = 16
NW = NC * NS

B = BATCH * SEQ
B_PER_W = B // NW
K = 8
NBUF = 3
RA = 1
NCHUNK = B_PER_W // K


def _gather_body(ids_hbm, table_hbm, out_hbm, idx_v, bufs, gsems, psems):
    wid = lax.axis_index("s") * NC + lax.axis_index("c")
    base = wid * B_PER_W

    pltpu.sync_copy(ids_hbm.at[pl.ds(wid * NCHUNK, NCHUNK)], idx_v)

    def gather_start(g, b):
        pltpu.async_copy(table_hbm.at[idx_v.at[g]], bufs[b], gsems[b])

    def gather_wait(b):
        pltpu.make_async_copy(table_hbm.at[idx_v.at[0]], bufs[b],
                              gsems[b]).wait()

    def put_start(g, b):
        pltpu.async_copy(bufs[b], out_hbm.at[pl.ds(base + g * K, K)], psems[b])

    def put_wait(b):
        pltpu.make_async_copy(bufs[b], out_hbm.at[pl.ds(base, K)],
                              psems[b]).wait()


    def iter_step(i, b, bnext, do_putwait, do_gather):
        if do_gather:
            if do_putwait:
                put_wait(bnext)
            gather_start(i + RA, bnext)
        gather_wait(b)
        put_start(i, b)

    for g in range(RA):
        gather_start(g, g % NBUF)

    H = NBUF - RA
    BULK = ((NCHUNK - RA) - H) // NBUF * NBUF
    for i in range(H):
        iter_step(i, i % NBUF, (i + RA) % NBUF, False, True)

    @pl.loop(H, H + BULK, step=NBUF)
    def _(i0):
        for j in range(NBUF):
            iter_step(i0 + j, (H + j) % NBUF, (H + j + RA) % NBUF, True, True)

    for i in range(H + BULK, NCHUNK - RA):
        iter_step(i, i % NBUF, (i + RA) % NBUF, True, True)
    for i in range(NCHUNK - RA, NCHUNK):
        iter_step(i, i % NBUF, None, False, False)
    for g in range(NCHUNK - NBUF, NCHUNK):
        put_wait(g % NBUF)


@jax.jit
def _embed_gather(ids, W):
    mesh = plsc.VectorSubcoreMesh(core_axis_name="c", subcore_axis_name="s")
    run = pl.kernel(
        _gather_body,
        out_type=jax.ShapeDtypeStruct((B, HIDDEN), jnp.float32),
        mesh=mesh,
        scratch_types=[
            pltpu.VMEM((NCHUNK, K), jnp.int32),
            [pltpu.VMEM((K, HIDDEN), jnp.float32) for _ in range(NBUF)],
            [pltpu.SemaphoreType.DMA for _ in range(NBUF)],
            [pltpu.SemaphoreType.DMA for _ in range(NBUF)],
        ],
    )
    return run(ids, W)


def kernel(inputs, W):
    ids = inputs[..., 0].reshape(B // K, K)
    attention_mask = inputs[..., 1]
    hidden = _embed_gather(ids, W)
    return hidden.reshape(BATCH, SEQ, HIDDEN), attention_mask
